# baseline (speedup 1.0000x reference)
import functools

import jax
import jax.numpy as jnp
from jax import lax
from jax.experimental import pallas as pl
from jax.experimental.pallas import tpu as pltpu
from jax.experimental.pallas import tpu_sc as plsc

_K = 8
_DD = 16
_D = 128
_N = 10000
_M = 320000
_NITER = 6

_NPAD = 10240
_NW = 32
_EPW = 10240
_MPAD = _NW * _EPW
_C = 128
_NCH = _EPW // _C
_RPT = _NPAD // 16


def _sc_edge_pass_body(u_hbm, xn_hbm, src_hbm, trg_hbm, out_hbm,
                       acc_sh, idx_v, z_v, ut_v,
                       sem_z, sem_u):
  cid = lax.axis_index("c")
  sid = lax.axis_index("s")
  wid = sid * 2 + cid

  zvec = jnp.zeros((16,), jnp.float32)

  def _zrow(i, _):
    for j in range(_D // 16):
      ut_v[i, pl.ds(j * 16, 16)] = zvec
    return 0

  lax.fori_loop(0, _C, _zrow, 0)
  for b in range(_RPT // _C):
    pltpu.sync_copy(ut_v, acc_sh.at[pl.ds(sid * _RPT + b * _C, _C)])
  plsc.subcore_barrier()

  lane = lax.broadcasted_iota(jnp.int32, (16,), 0)

  def _chunk(ci, _):
    pltpu.sync_copy(src_hbm.at[wid, ci], idx_v.at[0])
    pltpu.sync_copy(trg_hbm.at[wid, ci], idx_v.at[1])
    si = idx_v.at[0]
    ti = idx_v.at[1]
    cz = pltpu.async_copy(xn_hbm.at[si], z_v, sem_z)
    cu = pltpu.async_copy(u_hbm.at[ti], ut_v, sem_u)
    cz.wait()
    cu.wait()

    def _group(g, _):
      rows = lane + g * 16
      ps = []
      for k in range(_K):
        acc = None
        for j in range(_DD):
          col = jnp.full((16,), k * _DD + j, jnp.int32)
          zz = plsc.load_gather(z_v, [rows, col])
          uu = plsc.load_gather(ut_v, [rows, col])
          prod = zz * uu
          acc = prod if acc is None else acc + prod
        ps.append(acc)
      m = ps[0]
      for k in range(1, _K):
        m = jnp.maximum(m, ps[k])
      es = [jnp.exp(p - m) for p in ps]
      s = es[0]
      for k in range(1, _K):
        s = s + es[k]
      inv = 1.0 / s
      for k in range(_K):
        w = es[k] * inv
        for j in range(_DD):
          col = jnp.full((16,), k * _DD + j, jnp.int32)
          zz = plsc.load_gather(z_v, [rows, col])
          plsc.store_scatter(ut_v, [rows, col], zz * w)
      return 0

    lax.fori_loop(0, _C // 16, _group, 0)
    pltpu.sync_copy(ut_v, acc_sh.at[ti], add=True)
    return 0

  lax.fori_loop(0, _NCH, _chunk, 0)

  plsc.subcore_barrier()
  pltpu.sync_copy(acc_sh.at[pl.ds(sid * _RPT, _RPT)],
                  out_hbm.at[cid, pl.ds(sid * _RPT, _RPT)])


_sc_edge_pass = pl.kernel(
    _sc_edge_pass_body,
    out_type=jax.ShapeDtypeStruct((2, _NPAD, _D), jnp.float32),
    mesh=plsc.VectorSubcoreMesh(core_axis_name="c", subcore_axis_name="s"),
    scratch_types=[
        pltpu.VMEM_SHARED((_NPAD, _D), jnp.float32),
        pltpu.VMEM((2, _C), jnp.int32),
        pltpu.VMEM((_C, _D), jnp.float32),
        pltpu.VMEM((_C, _D), jnp.float32),
        pltpu.SemaphoreType.DMA,
        pltpu.SemaphoreType.DMA,
    ],
    compiler_params=pltpu.CompilerParams(needs_layout_passes=False),
    name="disen_edge_pass",
)


def _norm_chunks(v):
  parts = []
  for k in range(_K):
    s = v[:, k * _DD:(k + 1) * _DD]
    n = jnp.sqrt(jnp.sum(s * s, axis=1, keepdims=True))
    parts.append(s / jnp.maximum(n, 1e-12))
  return jnp.concatenate(parts, axis=1)


def _tc_init_body(x_ref, o_ref):
  o_ref[...] = _norm_chunks(x_ref[...])


def _tc_comb_body(p0_ref, p1_ref, xn_ref, o_ref):
  o_ref[...] = _norm_chunks(p0_ref[...] + p1_ref[...] + xn_ref[...])


_TCB = 256
_spec = pl.BlockSpec((_TCB, _D), lambda i: (i, 0))

_tc_init = pl.pallas_call(
    _tc_init_body,
    grid=(_NPAD // _TCB,),
    in_specs=[_spec],
    out_specs=_spec,
    out_shape=jax.ShapeDtypeStruct((_NPAD, _D), jnp.float32),
)

_tc_comb = pl.pallas_call(
    _tc_comb_body,
    grid=(_NPAD // _TCB,),
    in_specs=[_spec, _spec, _spec],
    out_specs=_spec,
    out_shape=jax.ShapeDtypeStruct((_NPAD, _D), jnp.float32),
)


@jax.jit
def kernel(x, edge_index):
  x = x.astype(jnp.float32)
  xp = jnp.pad(x, ((0, _NPAD - _N), (0, 0)))
  xn = _tc_init(xp)

  npad_e = _MPAD - _M
  pad_idx = _N + (jnp.arange(npad_e, dtype=jnp.int32) % (_NPAD - _N))
  srcp = jnp.concatenate([edge_index[0].astype(jnp.int32), pad_idx])
  trgp = jnp.concatenate([edge_index[1].astype(jnp.int32), pad_idx])
  srcp = srcp.reshape(_NW, _NCH, _C)
  trgp = trgp.reshape(_NW, _NCH, _C)

  u = xn
  for _ in range(_NITER):
    parts = _sc_edge_pass(u, xn, srcp, trgp)
    u = _tc_comb(parts[0], parts[1], xn)
  return u[:_N]
